# baseline (speedup 1.0000x reference)
import jax
import jax.numpy as jnp
from jax.experimental import pallas as pl
from jax.experimental.pallas import tpu as pltpu

N = 2048


def _body(srcT_ref, tgtT_ref, out_ref):
    srcT = srcT_ref[...]
    tgtT = tgtT_ref[...]
    tt = jnp.sum(tgtT * tgtT, axis=0, keepdims=True)
    ndot = jax.lax.dot_general(
        srcT * -2.0, tgtT, (((0,), (0,)), ((), ())),
        preferred_element_type=jnp.float32,
        precision=jax.lax.Precision.DEFAULT,
    )
    ss = jnp.sum(srcT * srcT, axis=0, keepdims=True)
    rmin = jnp.min(tt + ndot, axis=1)
    cmin = jnp.min(ndot + ss.T, axis=0)
    loss_s2t = (jnp.sum(rmin) + jnp.sum(ss)) / N
    loss_t2s = (jnp.sum(cmin) + jnp.sum(tt)) / N
    out_ref[0, 0] = loss_s2t + 0.8 * loss_t2s


def kernel(source_cloud, target_cloud):
    srcT = source_cloud.T
    tgtT = target_cloud.T
    out = pl.pallas_call(
        _body,
        out_specs=pl.BlockSpec(memory_space=pltpu.SMEM),
        out_shape=jax.ShapeDtypeStruct((1, 1), jnp.float32),
    )(srcT, tgtT)
    return out[0, 0]
